# baseline (speedup 1.0000x reference)
import functools

import jax
import jax.numpy as jnp
from jax import lax
from jax.experimental import pallas as pl
from jax.experimental.pallas import tpu as pltpu
from jax.experimental.pallas import tpu_sc as plsc

B = 4096
L = 16
NC = 2
NS = 16
NW = NC * NS
CHUNKS = B // L

SC_ROWS = 1024
ROWS = SC_ROWS // NW

_mesh = plsc.VectorSubcoreMesh(core_axis_name="c", subcore_axis_name="s")


def _tc_hinge_body(pred_ref, rank_ref, out_ref):
    flatp = pred_ref[...].reshape(1, B)
    flatr = rank_ref[...].reshape(1, B)

    def body(c1, carry):
        s, n = carry
        prow = pred_ref[c1]
        rrow = rank_ref[c1]
        m = rrow[:, None] < flatr
        d = jnp.maximum(1.0 + prow[:, None] - flatp, 0.0)
        s = s + jnp.sum(jnp.where(m, d * d, 0.0))
        n = n + jnp.sum(jnp.where(m, 1.0, 0.0))
        return s, n

    s, n = lax.fori_loop(SC_ROWS // 128, B // 128, body,
                         (jnp.float32(0.0), jnp.float32(0.0)))
    out_ref[...] = jnp.stack([jnp.full((128,), s), jnp.full((128,), n)])


def _tc_hinge(pred2d, rank2d):
    return pl.pallas_call(
        _tc_hinge_body,
        out_shape=jax.ShapeDtypeStruct((2, 128), jnp.float32),
    )(pred2d, rank2d)


@functools.partial(
    pl.kernel,
    mesh=_mesh,
    out_type=[
        jax.ShapeDtypeStruct((NW, L), jnp.float32),
        jax.ShapeDtypeStruct((NW, L), jnp.float32),
    ],
    scratch_types=[
        pltpu.VMEM((B,), jnp.float32),
        pltpu.VMEM((B,), jnp.int32),
        pltpu.VMEM((L,), jnp.float32),
        pltpu.VMEM((L,), jnp.float32),
    ],
)
def _rank_loss_partials(pred_hbm, rank_hbm, sum_hbm, cnt_hbm,
                        pred_v, rank_v, sacc_v, cacc_v):
    wid = lax.axis_index("s") * NC + lax.axis_index("c")
    pltpu.sync_copy(pred_hbm, pred_v)
    pltpu.sync_copy(rank_hbm, rank_v)
    base = wid * ROWS
    K = 8
    zero = jnp.zeros((L,), jnp.float32)
    izero = jnp.zeros((L,), jnp.int32)

    def rowchunk_body(jc, carry):
        acc, cnt = carry
        pjv = pred_v[pl.ds(base + jc * L, L)]
        rjv = rank_v[pl.ds(base + jc * L, L)]
        for g in range(L // K):
            pjs = [jnp.full((L,), pjv[g * K + t]) for t in range(K)]
            rjs = [jnp.full((L,), rjv[g * K + t]) for t in range(K)]

            def chunk_body(c, carry2):
                accs, cnts = carry2
                pv = pred_v[pl.ds(c * L, L)]
                rv = rank_v[pl.ds(c * L, L)]
                t1 = 1.0 - pv
                accs = list(accs)
                cnts = list(cnts)
                for t in range(K):
                    m = rv > rjs[t]
                    d = jnp.maximum(t1 + pjs[t], 0.0)
                    accs[t] = jnp.where(m, accs[t] + d * d, accs[t])
                    cnts[t] = jnp.where(m, cnts[t] + 1, cnts[t])
                return tuple(accs), tuple(cnts)

            accs, cnts = lax.fori_loop(
                0, CHUNKS, chunk_body,
                ((zero,) * K, (izero,) * K))
            for t in range(K):
                acc = acc + accs[t]
                cnt = cnt + cnts[t].astype(jnp.float32)
        return acc, cnt

    acc, cnt = lax.fori_loop(0, ROWS // L, rowchunk_body, (zero, zero))
    sacc_v[...] = acc
    cacc_v[...] = cnt
    pltpu.sync_copy(sacc_v, sum_hbm.at[wid])
    pltpu.sync_copy(cacc_v, cnt_hbm.at[wid])


def kernel(pred, rank_batch):
    rank_i32 = rank_batch.astype(jnp.int32)
    sums, cnts = _rank_loss_partials(pred, rank_i32)
    sn = _tc_hinge(pred.reshape(32, 128), rank_i32.reshape(32, 128))
    return (jnp.sum(sums) + sn[0, 0]) / (jnp.sum(cnts) + sn[1, 0])
